# baseline (speedup 1.0000x reference)
import functools

import jax
import jax.numpy as jnp
from jax import lax
from jax.experimental import pallas as pl
from jax.experimental.pallas import tpu as pltpu
from jax.experimental.pallas import tpu_sc as plsc

_info = plsc.get_sparse_core_info()
_NC = _info.num_cores
_NS = _info.num_subcores
_NW = _NC * _NS


def _make_gather(B, F, V, D):
    BLQ = 64
    chunk = BLQ * F
    b_per_w = B // _NW
    n_chunks = b_per_w // BLQ
    mesh = plsc.VectorSubcoreMesh(core_axis_name="c", subcore_axis_name="s")

    @functools.partial(
        pl.kernel,
        mesh=mesh,
        out_type=jax.ShapeDtypeStruct((B * F, D), jnp.float32),
        scratch_types=(
            [pltpu.VMEM((n_chunks * chunk,), jnp.int32)]
            + [pltpu.VMEM((chunk, D), jnp.float32)] * 2
            + [pltpu.SemaphoreType.DMA] * 4
        ),
        compiler_params=pltpu.CompilerParams(use_tc_tiling_on_sc=False),
    )
    def gather_kernel(idx_hbm, table_hbm, out_hbm, idx_all,
                      buf0, buf1, gs0, gs1, ws0, ws1):
        wid = lax.axis_index("s") * _NC + lax.axis_index("c")
        pltpu.sync_copy(
            idx_hbm.at[pl.ds(wid * n_chunks * chunk, n_chunks * chunk)],
            idx_all)

        bufs = (buf0, buf1)
        gsems = (gs0, gs1)
        wsems = (ws0, ws1)

        def gather(j):
            return pltpu.async_copy(
                table_hbm.at[idx_all.at[pl.ds(j * chunk, chunk)]],
                bufs[j % 2], gsems[j % 2])

        gh = [None] * n_chunks
        wh = [None] * n_chunks
        for j in range(min(2, n_chunks)):
            gh[j] = gather(j)
        for j in range(n_chunks):
            gh[j].wait()
            base = (wid * n_chunks + j) * chunk
            wh[j] = pltpu.async_copy(
                bufs[j % 2], out_hbm.at[pl.ds(base, chunk)], wsems[j % 2])
            if j + 2 < n_chunks:
                wh[j].wait()
                gh[j + 2] = gather(j + 2)
        for j in range(max(0, n_chunks - 2), n_chunks):
            if wh[j] is not None:
                wh[j].wait()

    return gather_kernel


def kernel(indices, table):
    B, F = indices.shape
    V, D = table.shape
    N = B * F
    chunk = 32 * F
    flat_idx = indices.reshape(N).astype(jnp.int32)
    out = _make_gather(B, F, V, D)(flat_idx, table)
    return out.reshape(B, F, D)


# speedup vs baseline: 1.0016x; 1.0016x over previous
import functools

import jax
import jax.numpy as jnp
from jax import lax
from jax.experimental import pallas as pl
from jax.experimental.pallas import tpu as pltpu
from jax.experimental.pallas import tpu_sc as plsc

_info = plsc.get_sparse_core_info()
_NC = _info.num_cores
_NS = _info.num_subcores
_NW = _NC * _NS


def _make_gather(B, F, V, D):
    BLQ = 64
    chunk = BLQ * F
    b_per_w = B // _NW
    n_chunks = b_per_w // BLQ
    mesh = plsc.VectorSubcoreMesh(core_axis_name="c", subcore_axis_name="s")

    @functools.partial(
        pl.kernel,
        mesh=mesh,
        out_type=jax.ShapeDtypeStruct((B * F, D), jnp.float32),
        scratch_types=(
            [pltpu.VMEM((n_chunks * chunk,), jnp.int32)]
            + [pltpu.VMEM((chunk, D), jnp.float32)] * 2
            + [pltpu.SemaphoreType.DMA] * 4
        ),
        compiler_params=pltpu.CompilerParams(use_tc_tiling_on_sc=False),
    )
    def gather_kernel(idx_hbm, table_hbm, out_hbm, idx_all,
                      buf0, buf1, gs0, gs1, ws0, ws1):
        wid = lax.axis_index("s") * _NC + lax.axis_index("c")
        pltpu.sync_copy(
            idx_hbm.at[pl.ds(wid * n_chunks * chunk, n_chunks * chunk)],
            idx_all)

        bufs = (buf0, buf1)
        gsems = (gs0, gs1)
        wsems = (ws0, ws1)

        def gather(j):
            return pltpu.async_copy(
                table_hbm.at[idx_all.at[pl.ds(j * chunk, chunk)]],
                bufs[j % 2], gsems[j % 2])

        gh = [None] * n_chunks
        wh = [None] * n_chunks
        for j in range(min(2, n_chunks)):
            gh[j] = gather(j)
        for j in range(n_chunks):
            gh[j].wait()
            base = (wid * n_chunks + j) * chunk
            wh[j] = pltpu.async_copy(
                bufs[j % 2], out_hbm.at[pl.ds(base, chunk)], wsems[j % 2])
            if j + 2 < n_chunks:
                wh[j].wait()
                gh[j + 2] = gather(j + 2)
        for j in range(max(0, n_chunks - 2), n_chunks):
            if wh[j] is not None:
                wh[j].wait()

    return gather_kernel


def kernel(indices, table):
    B, F = indices.shape
    V, D = table.shape
    N = B * F
    flat_idx = indices.reshape(N).astype(jnp.int32)
    out = _make_gather(B, F, V, D)(flat_idx, table)
    return out.reshape(B, F, D)
